# baseline (speedup 1.0000x reference)
import functools

import jax
import jax.numpy as jnp
from jax.experimental import pallas as pl
from jax.experimental.pallas import tpu as pltpu
from jax.experimental.pallas import tpu_sc as plsc

_N = 2048
_D = 768
_KB = 512
_NSC = 32
_GR = _N // _NSC


_TT = 256
_CW = 256


def _dist_argmin_body(nsteps, r_ref, cb_ref, a2_ref, b2_ref, idx_ref,
                      best_ref, jrun_ref):
    k = pl.program_id(1)
    nt = r_ref.shape[0]
    nchunks = _KB // _CW
    for tc in range(nt // _TT):
        tsl = pl.ds(tc * _TT, _TT)
        rt = r_ref[tsl, :]
        a2t = a2_ref[tsl, :]
        for c in range(nchunks):
            cbc = cb_ref[pl.ds(c * _CW, _CW), :]
            b2c = b2_ref[:, pl.ds(c * _CW, _CW)]
            ab = jax.lax.dot_general(
                rt, cbc,
                dimension_numbers=(((1,), (1,)), ((), ())),
                preferred_element_type=jnp.float32,
            )
            d2 = jnp.maximum((a2t + b2c) - 2.0 * ab, 0.0)
            cid = k * nchunks + c

            if c == 0:
                def _init(tsl=tsl, d2=d2, cid=cid):
                    best_ref[tsl, :] = d2
                    jrun_ref[tsl, :] = jnp.full(d2.shape, cid, jnp.int32)

                def _update(tsl=tsl, d2=d2, cid=cid):
                    better = d2 < best_ref[tsl, :]
                    jrun_ref[tsl, :] = jnp.where(
                        better, jnp.int32(cid), jrun_ref[tsl, :])
                    best_ref[tsl, :] = jnp.where(better, d2, best_ref[tsl, :])

                pl.when(k == 0)(_init)
                pl.when(k > 0)(_update)
            else:
                better = d2 < best_ref[tsl, :]
                jrun_ref[tsl, :] = jnp.where(
                    better, jnp.int32(cid), jrun_ref[tsl, :])
                best_ref[tsl, :] = jnp.where(better, d2, best_ref[tsl, :])

        def _extract(tsl=tsl):
            best = best_ref[tsl, :]
            lane = jax.lax.broadcasted_iota(jnp.int32, best.shape, 1)
            jfull = jrun_ref[tsl, :] * _CW + lane
            m = jnp.min(best, axis=1, keepdims=True)
            idx_ref[tsl, :] = jnp.min(
                jnp.where(best == m, jfull, jnp.int32(2**30)),
                axis=1, keepdims=True)

        pl.when(k == nsteps - 1)(_extract)


@functools.partial(jax.jit, static_argnames=("kk",))
def _dist_argmin(r, cb, a2, b2, kk):
    nt = _N // 2
    return pl.pallas_call(
        functools.partial(_dist_argmin_body, kk // _KB),
        grid=(2, kk // _KB),
        in_specs=[
            pl.BlockSpec((nt, _D), lambda i, k: (i, 0)),
            pl.BlockSpec((_KB, _D), lambda i, k: (k, 0)),
            pl.BlockSpec((nt, 1), lambda i, k: (i, 0)),
            pl.BlockSpec((1, _KB), lambda i, k: (0, k)),
        ],
        out_specs=pl.BlockSpec((nt, 1), lambda i, k: (i, 0)),
        out_shape=jax.ShapeDtypeStruct((_N, 1), jnp.int32),
        scratch_shapes=[
            pltpu.VMEM((nt, _CW), jnp.float32),
            pltpu.VMEM((nt, _CW), jnp.int32),
        ],
        compiler_params=pltpu.CompilerParams(
            dimension_semantics=("parallel", "arbitrary"),
        ),
    )(r, cb, a2, b2)


def _sc_gather(cb, idx):
    mesh = plsc.VectorSubcoreMesh(core_axis_name="core", subcore_axis_name="subcore")

    @pl.kernel(
        out_type=jax.ShapeDtypeStruct((_N, _D), jnp.float32),
        mesh=mesh,
        scratch_types=[
            pltpu.VMEM((_GR, _D), jnp.float32),
            pltpu.VMEM((1, 128), jnp.int32),
        ],
    )
    def kern(cb_hbm, i_hbm, o_hbm, qbuf, ibuf):
        u = jax.lax.axis_index("core") * 16 + jax.lax.axis_index("subcore")
        pltpu.sync_copy(i_hbm.at[pl.ds(u // 2, 1)], ibuf)
        off = (u % 2) * _GR
        pltpu.sync_copy(cb_hbm.at[ibuf.at[0, pl.ds(off, _GR)]], qbuf)
        pltpu.sync_copy(qbuf, o_hbm.at[pl.ds(u * _GR, _GR)])

    return kern(cb, idx)


def kernel(x, codebook_0, codebook_1, codebook_2, codebook_3):
    codebooks = [codebook_0, codebook_1, codebook_2, codebook_3]
    b, t, d = x.shape
    x2d = x.reshape(-1, d)
    residual = x2d
    a2 = jnp.sum(residual * residual, axis=1, keepdims=True)
    all_indices = []
    commit_ssq = []
    for cb in codebooks:
        b2 = jnp.sum(cb * cb, axis=1)[None, :]
        idx = _dist_argmin(residual, cb, a2, b2, cb.shape[0])
        q = _sc_gather(cb, idx.reshape(16, 128))
        q_st = residual + (q - residual)
        residual = residual - q_st
        a2 = jnp.sum(residual * residual, axis=1, keepdims=True)
        commit_ssq.append(jnp.sum(a2))
        all_indices.append(idx.reshape(b, t))
    quantized = x2d - residual
    total_commit = (
        (commit_ssq[0] + commit_ssq[1] + commit_ssq[2] + commit_ssq[3])
        * (0.25 / (b * t * d))
    ).astype(jnp.float32)
    all_indices = jnp.stack(all_indices, axis=-1)
    return quantized.reshape(b, t, d), all_indices, total_commit


# speedup vs baseline: 1.3825x; 1.3825x over previous
import functools

import jax
import jax.numpy as jnp
from jax.experimental import pallas as pl
from jax.experimental.pallas import tpu as pltpu
from jax.experimental.pallas import tpu_sc as plsc

_N = 2048
_D = 768
_KB = 512
_NSC = 32
_GR = _N // _NSC


def _dist_argmin_body(r_ref, cb_ref, a2_ref, b2_ref, idx_ref, best_ref):
    k = pl.program_id(1)
    ab = jax.lax.dot_general(
        r_ref[...], cb_ref[...],
        dimension_numbers=(((1,), (1,)), ((), ())),
        preferred_element_type=jnp.float32,
    )
    s = a2_ref[...] + b2_ref[...]
    d2 = jnp.maximum(s - 2.0 * ab, 0.0)
    m = jnp.min(d2, axis=1, keepdims=True)
    j = jax.lax.broadcasted_iota(jnp.int32, d2.shape, 1)
    lidx = jnp.min(jnp.where(d2 == m, j, jnp.int32(2**30)), axis=1, keepdims=True)
    gidx = lidx + k * _KB

    @pl.when(k == 0)
    def _():
        best_ref[...] = m
        idx_ref[...] = gidx

    @pl.when(k > 0)
    def _():
        better = m < best_ref[...]
        idx_ref[...] = jnp.where(better, gidx, idx_ref[...])
        best_ref[...] = jnp.where(better, m, best_ref[...])


@functools.partial(jax.jit, static_argnames=("kk",))
def _dist_argmin(r, cb, a2, b2, kk):
    nt = _N // 2
    return pl.pallas_call(
        _dist_argmin_body,
        grid=(2, kk // _KB),
        in_specs=[
            pl.BlockSpec((nt, _D), lambda i, k: (i, 0)),
            pl.BlockSpec((_KB, _D), lambda i, k: (k, 0)),
            pl.BlockSpec((nt, 1), lambda i, k: (i, 0)),
            pl.BlockSpec((1, _KB), lambda i, k: (0, k)),
        ],
        out_specs=pl.BlockSpec((nt, 1), lambda i, k: (i, 0)),
        out_shape=jax.ShapeDtypeStruct((_N, 1), jnp.int32),
        scratch_shapes=[pltpu.VMEM((nt, 1), jnp.float32)],
        compiler_params=pltpu.CompilerParams(
            dimension_semantics=("parallel", "arbitrary"),
        ),
    )(r.astype(jnp.bfloat16), cb.astype(jnp.bfloat16), a2, b2)


def _sc_gather(cb, idx):
    mesh = plsc.VectorSubcoreMesh(core_axis_name="core", subcore_axis_name="subcore")

    @pl.kernel(
        out_type=jax.ShapeDtypeStruct((_N, _D), jnp.float32),
        mesh=mesh,
        scratch_types=[
            pltpu.VMEM((_GR, _D), jnp.float32),
            pltpu.VMEM((1, 128), jnp.int32),
        ],
    )
    def kern(cb_hbm, i_hbm, o_hbm, qbuf, ibuf):
        u = jax.lax.axis_index("core") * 16 + jax.lax.axis_index("subcore")
        pltpu.sync_copy(i_hbm.at[pl.ds(u // 2, 1)], ibuf)
        off = (u % 2) * _GR
        pltpu.sync_copy(cb_hbm.at[ibuf.at[0, pl.ds(off, _GR)]], qbuf)
        pltpu.sync_copy(qbuf, o_hbm.at[pl.ds(u * _GR, _GR)])

    return kern(cb, idx)


def kernel(x, codebook_0, codebook_1, codebook_2, codebook_3):
    codebooks = [codebook_0, codebook_1, codebook_2, codebook_3]
    b, t, d = x.shape
    x2d = x.reshape(-1, d)
    residual = x2d
    a2 = jnp.sum(residual * residual, axis=1, keepdims=True)
    all_indices = []
    commit_ssq = []
    for cb in codebooks:
        b2 = jnp.sum(cb * cb, axis=1)[None, :]
        idx = _dist_argmin(residual, cb, a2, b2, cb.shape[0])
        q = _sc_gather(cb, idx.reshape(16, 128))
        q_st = residual + (q - residual)
        residual = residual - q_st
        a2 = jnp.sum(residual * residual, axis=1, keepdims=True)
        commit_ssq.append(jnp.sum(a2))
        all_indices.append(idx.reshape(b, t))
    quantized = x2d - residual
    total_commit = (
        (commit_ssq[0] + commit_ssq[1] + commit_ssq[2] + commit_ssq[3])
        * (0.25 / (b * t * d))
    ).astype(jnp.float32)
    all_indices = jnp.stack(all_indices, axis=-1)
    return quantized.reshape(b, t, d), all_indices, total_commit


# speedup vs baseline: 1.4779x; 1.0690x over previous
import functools

import jax
import jax.numpy as jnp
from jax.experimental import pallas as pl
from jax.experimental.pallas import tpu as pltpu
from jax.experimental.pallas import tpu_sc as plsc

_N = 2048
_D = 768
_KB = 512
_NSC = 32
_GR = _N // _NSC


def _dist_argmin_body(r_ref, cb_ref, a2_ref, b2_ref, idx_ref, ssq_ref, best_ref):
    k = pl.program_id(1)
    ab = jax.lax.dot_general(
        r_ref[...], cb_ref[...],
        dimension_numbers=(((1,), (1,)), ((), ())),
        preferred_element_type=jnp.float32,
    )
    s = a2_ref[...] + b2_ref[...]
    d2 = jnp.maximum(s - 2.0 * ab, 0.0)
    m = jnp.min(d2, axis=1, keepdims=True)
    j = jax.lax.broadcasted_iota(jnp.int32, d2.shape, 1)
    lidx = jnp.min(jnp.where(d2 == m, j, jnp.int32(2**30)), axis=1, keepdims=True)
    gidx = lidx + k * _KB

    @pl.when(k == 0)
    def _():
        best_ref[...] = m
        idx_ref[...] = gidx
        ssq_ref[...] = jnp.full(ssq_ref.shape, jnp.sum(a2_ref[...]), jnp.float32)

    @pl.when(k > 0)
    def _():
        better = m < best_ref[...]
        idx_ref[...] = jnp.where(better, gidx, idx_ref[...])
        best_ref[...] = jnp.where(better, m, best_ref[...])


@functools.partial(jax.jit, static_argnames=("kk",))
def _dist_argmin(r, cb, a2, b2, kk):
    nt = _N // 2
    return pl.pallas_call(
        _dist_argmin_body,
        grid=(2, kk // _KB),
        in_specs=[
            pl.BlockSpec((nt, _D), lambda i, k: (i, 0)),
            pl.BlockSpec((_KB, _D), lambda i, k: (k, 0)),
            pl.BlockSpec((nt, 1), lambda i, k: (i, 0)),
            pl.BlockSpec((1, _KB), lambda i, k: (0, k)),
        ],
        out_specs=[
            pl.BlockSpec((nt, 1), lambda i, k: (i, 0)),
            pl.BlockSpec((8, 128), lambda i, k: (i, 0)),
        ],
        out_shape=[
            jax.ShapeDtypeStruct((_N, 1), jnp.int32),
            jax.ShapeDtypeStruct((16, 128), jnp.float32),
        ],
        scratch_shapes=[pltpu.VMEM((nt, 1), jnp.float32)],
        compiler_params=pltpu.CompilerParams(
            dimension_semantics=("parallel", "arbitrary"),
        ),
    )(r, cb, a2, b2)


def _sc_gather(cb, idx):
    mesh = plsc.VectorSubcoreMesh(core_axis_name="core", subcore_axis_name="subcore")

    half = _GR // 2

    @pl.kernel(
        out_type=jax.ShapeDtypeStruct((_N, _D), jnp.float32),
        mesh=mesh,
        scratch_types=[
            pltpu.VMEM((_GR, _D), jnp.float32),
            pltpu.VMEM((1, 128), jnp.int32),
            pltpu.SemaphoreType.DMA,
            pltpu.SemaphoreType.DMA,
            pltpu.SemaphoreType.DMA,
        ],
    )
    def kern(cb_hbm, i_hbm, o_hbm, qbuf, ibuf, sg0, sg1, so0):
        u = jax.lax.axis_index("core") * 16 + jax.lax.axis_index("subcore")
        pltpu.sync_copy(i_hbm.at[pl.ds(u // 2, 1)], ibuf)
        off = (u % 2) * _GR
        g0 = pltpu.make_async_copy(
            cb_hbm.at[ibuf.at[0, pl.ds(off, half)]],
            qbuf.at[pl.ds(0, half)], sg0)
        g0.start()
        g1 = pltpu.make_async_copy(
            cb_hbm.at[ibuf.at[0, pl.ds(off + half, half)]],
            qbuf.at[pl.ds(half, half)], sg1)
        g0.wait()
        o0 = pltpu.make_async_copy(
            qbuf.at[pl.ds(0, half)], o_hbm.at[pl.ds(u * _GR, half)], so0)
        o0.start()
        g1.start()
        g1.wait()
        pltpu.sync_copy(qbuf.at[pl.ds(half, half)],
                        o_hbm.at[pl.ds(u * _GR + half, half)])
        o0.wait()

    return kern(cb, idx)


def kernel(x, codebook_0, codebook_1, codebook_2, codebook_3):
    codebooks = [codebook_0, codebook_1, codebook_2, codebook_3]
    b, t, d = x.shape
    x2d = x.reshape(-1, d)
    residual = x2d
    a2 = jnp.sum(residual * residual, axis=1, keepdims=True)
    all_indices = []
    commit_ssq = []
    for cb in codebooks:
        b2 = jnp.sum(cb * cb, axis=1)[None, :]
        idx, ssq = _dist_argmin(residual, cb, a2, b2, cb.shape[0])
        commit_ssq.append(ssq[0, 0] + ssq[8, 0])
        q = _sc_gather(cb, idx.reshape(16, 128))
        q_st = residual + (q - residual)
        residual = residual - q_st
        a2 = jnp.sum(residual * residual, axis=1, keepdims=True)
        all_indices.append(idx.reshape(b, t))
    quantized = x2d - residual
    total_commit = (
        (commit_ssq[1] + commit_ssq[2] + commit_ssq[3] + jnp.sum(a2))
        * (0.25 / (b * t * d))
    ).astype(jnp.float32)
    all_indices = jnp.stack(all_indices, axis=-1)
    return quantized.reshape(b, t, d), all_indices, total_commit


# speedup vs baseline: 1.6119x; 1.0907x over previous
import functools

import jax
import jax.numpy as jnp
from jax.experimental import pallas as pl
from jax.experimental.pallas import tpu as pltpu
from jax.experimental.pallas import tpu_sc as plsc

_N = 2048
_D = 768
_KB = 1024
_NSC = 32
_GR = _N // _NSC


def _dist_argmin_body(r_ref, cb_ref, a2_ref, b2_ref, idx_ref, ssq_ref, best_ref):
    k = pl.program_id(1)
    ab = jax.lax.dot_general(
        r_ref[...], cb_ref[...],
        dimension_numbers=(((1,), (1,)), ((), ())),
        preferred_element_type=jnp.float32,
    )
    s = a2_ref[...] + b2_ref[...]
    d2 = jnp.maximum(s - 2.0 * ab, 0.0)
    m = jnp.min(d2, axis=1, keepdims=True)
    j = jax.lax.broadcasted_iota(jnp.int32, d2.shape, 1)
    lidx = jnp.min(jnp.where(d2 == m, j, jnp.int32(2**30)), axis=1, keepdims=True)
    gidx = lidx + k * _KB

    @pl.when(k == 0)
    def _():
        best_ref[...] = m
        idx_ref[...] = gidx
        ssq_ref[...] = jnp.full(ssq_ref.shape, jnp.sum(a2_ref[...]), jnp.float32)

    @pl.when(k > 0)
    def _():
        better = m < best_ref[...]
        idx_ref[...] = jnp.where(better, gidx, idx_ref[...])
        best_ref[...] = jnp.where(better, m, best_ref[...])


@functools.partial(jax.jit, static_argnames=("kk",))
def _dist_argmin(r, cb, a2, b2, kk):
    nt = _N // 2
    return pl.pallas_call(
        _dist_argmin_body,
        grid=(2, kk // _KB),
        in_specs=[
            pl.BlockSpec((nt, _D), lambda i, k: (i, 0)),
            pl.BlockSpec((_KB, _D), lambda i, k: (k, 0)),
            pl.BlockSpec((nt, 1), lambda i, k: (i, 0)),
            pl.BlockSpec((1, _KB), lambda i, k: (0, k)),
        ],
        out_specs=[
            pl.BlockSpec((nt, 1), lambda i, k: (i, 0)),
            pl.BlockSpec((8, 128), lambda i, k: (i, 0)),
        ],
        out_shape=[
            jax.ShapeDtypeStruct((_N, 1), jnp.int32),
            jax.ShapeDtypeStruct((16, 128), jnp.float32),
        ],
        scratch_shapes=[pltpu.VMEM((nt, 1), jnp.float32)],
        compiler_params=pltpu.CompilerParams(
            dimension_semantics=("parallel", "arbitrary"),
        ),
    )(r, cb, a2, b2)


def _sc_gather(cb, idx):
    mesh = plsc.VectorSubcoreMesh(core_axis_name="core", subcore_axis_name="subcore")

    @pl.kernel(
        out_type=jax.ShapeDtypeStruct((_N, _D), jnp.float32),
        mesh=mesh,
        scratch_types=[
            pltpu.VMEM((_GR, _D), jnp.float32),
            pltpu.VMEM((1, 128), jnp.int32),
        ],
    )
    def kern(cb_hbm, i_hbm, o_hbm, qbuf, ibuf):
        u = jax.lax.axis_index("core") * 16 + jax.lax.axis_index("subcore")
        pltpu.sync_copy(i_hbm.at[pl.ds(u // 2, 1)], ibuf)
        off = (u % 2) * _GR
        pltpu.sync_copy(cb_hbm.at[ibuf.at[0, pl.ds(off, _GR)]], qbuf)
        pltpu.sync_copy(qbuf, o_hbm.at[pl.ds(u * _GR, _GR)])

    return kern(cb, idx)


def kernel(x, codebook_0, codebook_1, codebook_2, codebook_3):
    codebooks = [codebook_0, codebook_1, codebook_2, codebook_3]
    b, t, d = x.shape
    x2d = x.reshape(-1, d)
    residual = x2d
    a2 = jnp.sum(residual * residual, axis=1, keepdims=True)
    all_indices = []
    commit_ssq = []
    for cb in codebooks:
        b2 = jnp.sum(cb * cb, axis=1)[None, :]
        idx, ssq = _dist_argmin(residual, cb, a2, b2, cb.shape[0])
        commit_ssq.append(ssq[0, 0] + ssq[8, 0])
        q = _sc_gather(cb, idx.reshape(16, 128))
        q_st = residual + (q - residual)
        residual = residual - q_st
        a2 = jnp.sum(residual * residual, axis=1, keepdims=True)
        all_indices.append(idx.reshape(b, t))
    quantized = x2d - residual
    total_commit = (
        (commit_ssq[1] + commit_ssq[2] + commit_ssq[3] + jnp.sum(a2))
        * (0.25 / (b * t * d))
    ).astype(jnp.float32)
    all_indices = jnp.stack(all_indices, axis=-1)
    return quantized.reshape(b, t, d), all_indices, total_commit


# speedup vs baseline: 1.6705x; 1.0364x over previous
import functools

import jax
import jax.numpy as jnp
from jax.experimental import pallas as pl
from jax.experimental.pallas import tpu as pltpu
from jax.experimental.pallas import tpu_sc as plsc

_N = 2048
_D = 768
_KB = 2048
_NSC = 32
_GR = _N // _NSC


def _dist_argmin_body(kb, r_ref, cb_ref, a2_ref, b2_ref, idx_ref, ssq_ref, best_ref):
    k = pl.program_id(1)
    ab = jax.lax.dot_general(
        r_ref[...], cb_ref[...],
        dimension_numbers=(((1,), (1,)), ((), ())),
        preferred_element_type=jnp.float32,
    )
    s = a2_ref[...] + b2_ref[...]
    d2 = jnp.maximum(s - 2.0 * ab, 0.0)
    m = jnp.min(d2, axis=1, keepdims=True)
    j = jax.lax.broadcasted_iota(jnp.int32, d2.shape, 1)
    lidx = jnp.min(jnp.where(d2 == m, j, jnp.int32(2**30)), axis=1, keepdims=True)
    gidx = lidx + k * kb

    @pl.when(k == 0)
    def _():
        best_ref[...] = m
        idx_ref[...] = gidx
        ssq_ref[...] = jnp.full(ssq_ref.shape, jnp.sum(a2_ref[...]), jnp.float32)

    @pl.when(k > 0)
    def _():
        better = m < best_ref[...]
        idx_ref[...] = jnp.where(better, gidx, idx_ref[...])
        best_ref[...] = jnp.where(better, m, best_ref[...])


@functools.partial(jax.jit, static_argnames=("kk",))
def _dist_argmin(r, cb, a2, b2, kk):
    nt = _N // 2
    kb = min(kk, _KB)
    return pl.pallas_call(
        functools.partial(_dist_argmin_body, kb),
        grid=(2, kk // kb),
        in_specs=[
            pl.BlockSpec((nt, _D), lambda i, k: (i, 0)),
            pl.BlockSpec((kb, _D), lambda i, k: (k, 0)),
            pl.BlockSpec((nt, 1), lambda i, k: (i, 0)),
            pl.BlockSpec((1, kb), lambda i, k: (0, k)),
        ],
        out_specs=[
            pl.BlockSpec((nt, 1), lambda i, k: (i, 0)),
            pl.BlockSpec((8, 128), lambda i, k: (i, 0)),
        ],
        out_shape=[
            jax.ShapeDtypeStruct((_N, 1), jnp.int32),
            jax.ShapeDtypeStruct((16, 128), jnp.float32),
        ],
        scratch_shapes=[pltpu.VMEM((nt, 1), jnp.float32)],
        compiler_params=pltpu.CompilerParams(
            dimension_semantics=("parallel", "arbitrary"),
        ),
    )(r, cb, a2, b2)


def _sc_gather(cb, idx):
    mesh = plsc.VectorSubcoreMesh(core_axis_name="core", subcore_axis_name="subcore")

    @pl.kernel(
        out_type=jax.ShapeDtypeStruct((_N, _D), jnp.float32),
        mesh=mesh,
        scratch_types=[
            pltpu.VMEM((_GR, _D), jnp.float32),
            pltpu.VMEM((1, 128), jnp.int32),
        ],
    )
    def kern(cb_hbm, i_hbm, o_hbm, qbuf, ibuf):
        u = jax.lax.axis_index("core") * 16 + jax.lax.axis_index("subcore")
        pltpu.sync_copy(i_hbm.at[pl.ds(u // 2, 1)], ibuf)
        off = (u % 2) * _GR
        pltpu.sync_copy(cb_hbm.at[ibuf.at[0, pl.ds(off, _GR)]], qbuf)
        pltpu.sync_copy(qbuf, o_hbm.at[pl.ds(u * _GR, _GR)])

    return kern(cb, idx)


def kernel(x, codebook_0, codebook_1, codebook_2, codebook_3):
    codebooks = [codebook_0, codebook_1, codebook_2, codebook_3]
    b, t, d = x.shape
    x2d = x.reshape(-1, d)
    residual = x2d
    a2 = jnp.sum(residual * residual, axis=1, keepdims=True)
    all_indices = []
    commit_ssq = []
    for cb in codebooks:
        b2 = jnp.sum(cb * cb, axis=1)[None, :]
        idx, ssq = _dist_argmin(residual, cb, a2, b2, cb.shape[0])
        commit_ssq.append(ssq[0, 0] + ssq[8, 0])
        q = _sc_gather(cb, idx.reshape(16, 128))
        q_st = residual + (q - residual)
        residual = residual - q_st
        a2 = jnp.sum(residual * residual, axis=1, keepdims=True)
        all_indices.append(idx.reshape(b, t))
    quantized = x2d - residual
    total_commit = (
        (commit_ssq[1] + commit_ssq[2] + commit_ssq[3] + jnp.sum(a2))
        * (0.25 / (b * t * d))
    ).astype(jnp.float32)
    all_indices = jnp.stack(all_indices, axis=-1)
    return quantized.reshape(b, t, d), all_indices, total_commit
